# baseline (speedup 1.0000x reference)
import functools

import jax
import jax.numpy as jnp
from jax import lax
from jax.experimental import pallas as pl
from jax.experimental.pallas import tpu as pltpu
from jax.experimental.pallas import tpu_sc as plsc

_MARGIN = 1.0
_N_NEG = 10
_B, _L, _S = 4, 4096, 4096
_P = 8192
_LANES = 16
_NC, _NS = 2, 16
_NW = _NC * _NS
_NVEC = _S // _LANES
_CH = 8
_RPW0 = 288
_RPW1 = 224
_PPAD = _NS * _RPW0 + _NS * _RPW1 + _RPW0 + _LANES
_NEGINF = -3.0e38
_MASKVAL = -1000000000.0
_SCALE = 1.0 / (_P * _N_NEG)


def _sortd(x):
    k, _ = plsc.sort_key_val(x, x, descending=True)
    return k


def _rev(x):
    return lax.rev(x, (0,))


def _body(table, rowidx, jids, mref, out, idx_v, jv, rowbuf_a, rowbuf_b,
          cand, accv, m_v, sem_a, sem_b):
    cid = lax.axis_index("c")
    sid = lax.axis_index("s")
    wid = sid * _NC + cid
    base = jnp.where(cid == 1, sid * _RPW0, _NS * _RPW0 + sid * _RPW1)
    npairs = jnp.where(cid == 1, _RPW0 // (2 * _CH), _RPW1 // (2 * _CH))
    pltpu.sync_copy(rowidx.at[pl.ds(base, _RPW0)], idx_v)
    pltpu.sync_copy(jids.at[pl.ds(base, _RPW0 + _LANES)], jv)
    pltpu.sync_copy(mref, m_v)

    ninf = jnp.full((_LANES,), _NEGINF, jnp.float32)
    izero = jnp.zeros((_LANES,), jnp.int32)
    ione = jnp.full((_LANES,), 1, jnp.int32)
    i16 = jnp.full((_LANES,), _LANES, jnp.int32)
    lanes = lax.iota(jnp.int32, _LANES)
    lane3 = lanes == 3
    row8 = lanes < _CH
    m0c = m_v[0]
    m1c = m_v[1]

    def process_chunk(rowbuf, c, acc):
        jvec = jv[pl.ds(c * _CH, _LANES)]
        posv = plsc.load_gather(rowbuf, [lanes, jvec], mask=row8)
        plsc.store_scatter(rowbuf, [lanes, jvec],
                           jnp.full((_LANES,), _MASKVAL, jnp.float32),
                           mask=row8)

        def pass1(rl):
            def p1(i, c2):
                e1, e2 = c2
                v = rowbuf[rl, pl.ds(i * _LANES, _LANES)]
                e2 = jnp.maximum(e2, jnp.minimum(e1, v))
                e1 = jnp.maximum(e1, v)
                return e1, e2

            return lax.fori_loop(0, _NVEC, p1, (ninf, ninf), unroll=16)

        def tau_of(e1, e2):
            lo = jnp.minimum(_sortd(e1), _rev(_sortd(e2)))
            return lax.reduce_max(jnp.where(lane3, _sortd(lo), _NEGINF),
                                  axes=(0,))

        def pass2(rl, tau):
            base = lanes + rl * _S

            @plsc.parallel_loop(0, _NVEC, unroll=16, carry=base)
            def oidx(i, oidx):
                v = rowbuf[rl, pl.ds(i * _LANES, _LANES)]
                m = v >= tau
                plsc.store_scatter(cand, [oidx], v, mask=m)
                return oidx + jnp.where(m, i16, izero)

            return lax.shift_right_logical(oidx - base, 4)

        def merge1(s, rl, t0, t1, cnt):
            v = jnp.where(cnt > s, cand[pl.ds(rl * _S + s * _LANES, _LANES)],
                          _NEGINF)
            sv = _sortd(v)
            a = _sortd(jnp.maximum(t1, _rev(sv)))
            ra = _rev(a)
            return _sortd(jnp.maximum(t0, ra)), _sortd(jnp.minimum(t0, ra))

        def hinge(rl, t0, t1):
            d = _MARGIN - posv[rl]
            return (m0c * jnp.maximum(d + t0, 0.0)
                    + m1c * jnp.maximum(d + t1, 0.0))

        es = [pass1(rl) for rl in range(_CH)]
        taus = [tau_of(e1, e2) for (e1, e2) in es]
        cnts = [pass2(rl, taus[rl]) for rl in range(_CH)]
        cmax = cnts[0]
        for rl in range(1, _CH):
            cmax = jnp.maximum(cmax, cnts[rl])
        nmax = lax.reduce_max(cmax, axes=(0,))

        def mrg8(s, ts):
            return tuple(
                v for rl in range(_CH)
                for v in merge1(s, rl, ts[2 * rl], ts[2 * rl + 1], cnts[rl]))

        ts = lax.fori_loop(0, nmax, mrg8, (ninf,) * (2 * _CH))
        for rl in range(_CH):
            acc = acc + hinge(rl, ts[2 * rl], ts[2 * rl + 1])
        return acc

    nchunk = npairs * 2
    pltpu.async_copy(table.at[idx_v.at[pl.ds(0, _CH)]], rowbuf_a, sem_a)
    pltpu.async_copy(table.at[idx_v.at[pl.ds(_CH, _CH)]], rowbuf_b, sem_b)

    def pair_body(cc, acc):
        c0 = cc * 2
        c1 = c0 + 1
        pltpu.make_async_copy(table.at[idx_v.at[pl.ds(c0 * _CH, _CH)]],
                              rowbuf_a, sem_a).wait()
        acc = process_chunk(rowbuf_a, c0, acc)

        @pl.when(c0 + 2 < nchunk)
        def _():
            pltpu.async_copy(table.at[idx_v.at[pl.ds((c0 + 2) * _CH, _CH)]],
                             rowbuf_a, sem_a)

        pltpu.make_async_copy(table.at[idx_v.at[pl.ds(c1 * _CH, _CH)]],
                              rowbuf_b, sem_b).wait()
        acc = process_chunk(rowbuf_b, c1, acc)

        @pl.when(c1 + 2 < nchunk)
        def _():
            pltpu.async_copy(table.at[idx_v.at[pl.ds((c1 + 2) * _CH, _CH)]],
                             rowbuf_b, sem_b)

        return acc

    acc = lax.fori_loop(0, npairs, pair_body,
                        jnp.zeros((_LANES,), jnp.float32))
    accv[...] = acc * _SCALE
    pltpu.sync_copy(accv, out.at[wid])


_triplet_sc = functools.partial(
    pl.kernel,
    out_type=jax.ShapeDtypeStruct((_NW, _LANES), jnp.float32),
    mesh=plsc.VectorSubcoreMesh(
        core_axis_name="c", subcore_axis_name="s",
        num_cores=_NC, num_subcores=_NS),
    scratch_types=[
        pltpu.VMEM((_RPW0,), jnp.int32),
        pltpu.VMEM((_RPW0 + _LANES,), jnp.int32),
        pltpu.VMEM((_CH, _S), jnp.float32),
        pltpu.VMEM((_CH, _S), jnp.float32),
        pltpu.VMEM((_CH * _S,), jnp.float32),
        pltpu.VMEM((_LANES,), jnp.float32),
        pltpu.VMEM((2, _LANES), jnp.float32),
        pltpu.SemaphoreType.DMA,
        pltpu.SemaphoreType.DMA,
    ],
    compiler_params=pltpu.CompilerParams(needs_layout_passes=False),
)(_body)


def kernel(sim_matrix, b_ids, i_ids, j_ids):
    table = sim_matrix.reshape(_B * _L, _S)
    pad = jnp.zeros((_PPAD - _P,), jnp.int32)
    rowidx = jnp.concatenate([(b_ids * _L + i_ids).astype(jnp.int32), pad])
    jr = jnp.concatenate([j_ids.astype(jnp.int32), pad])
    perm = jax.random.permutation(jax.random.key(42), 2 * _N_NEG)[:_N_NEG]
    masks = (jnp.arange(2 * _LANES)[None, :] == perm[:, None]).astype(
        jnp.float32).sum(axis=0).reshape(2, _LANES)
    out = _triplet_sc(table, rowidx, jr, masks)
    return jnp.sum(out)


# speedup vs baseline: 1.0769x; 1.0769x over previous
import functools

import jax
import jax.numpy as jnp
from jax import lax
from jax.experimental import pallas as pl
from jax.experimental.pallas import tpu as pltpu
from jax.experimental.pallas import tpu_sc as plsc

_MARGIN = 1.0
_N_NEG = 10
_B, _L, _S = 4, 4096, 4096
_P = 8192
_LANES = 16
_NC, _NS = 2, 16
_NW = _NC * _NS
_NVEC = _S // _LANES
_CH = 8
_RPW0 = 256
_RPW1 = 256
_PPAD = _NS * _RPW0 + _NS * _RPW1 + _RPW0 + _LANES
_NEGINF = -3.0e38
_MASKVAL = -1000000000.0
_SCALE = 1.0 / (_P * _N_NEG)


def _sortd(x):
    k, _ = plsc.sort_key_val(x, x, descending=True)
    return k


def _rev(x):
    return lax.rev(x, (0,))


def _body(table, rowidx, jids, mref, out, idx_v, jv, rowbuf_a, rowbuf_b,
          cand, accv, m_v, sem_a, sem_b):
    cid = lax.axis_index("c")
    sid = lax.axis_index("s")
    wid = sid * _NC + cid
    base = jnp.where(cid == 1, sid * _RPW0, _NS * _RPW0 + sid * _RPW1)
    npairs = jnp.where(cid == 1, _RPW0 // (2 * _CH), _RPW1 // (2 * _CH))
    pltpu.sync_copy(rowidx.at[pl.ds(base, _RPW0)], idx_v)
    pltpu.sync_copy(jids.at[pl.ds(base, _RPW0 + _LANES)], jv)
    pltpu.sync_copy(mref, m_v)

    ninf = jnp.full((_LANES,), _NEGINF, jnp.float32)
    izero = jnp.zeros((_LANES,), jnp.int32)
    ione = jnp.full((_LANES,), 1, jnp.int32)
    i16 = jnp.full((_LANES,), _LANES, jnp.int32)
    lanes = lax.iota(jnp.int32, _LANES)
    lane3 = lanes == 3
    row8 = lanes < _CH
    m0c = m_v[0]
    m1c = m_v[1]

    def process_chunk(rowbuf, c, acc):
        jvec = jv[pl.ds(c * _CH, _LANES)]
        posv = plsc.load_gather(rowbuf, [lanes, jvec], mask=row8)
        plsc.store_scatter(rowbuf, [lanes, jvec],
                           jnp.full((_LANES,), _MASKVAL, jnp.float32),
                           mask=row8)

        def pass1(rl):
            def p1(i, c2):
                e1, e2 = c2
                v = rowbuf[rl, pl.ds(i * _LANES, _LANES)]
                e2 = jnp.maximum(e2, jnp.minimum(e1, v))
                e1 = jnp.maximum(e1, v)
                return e1, e2

            return lax.fori_loop(0, _NVEC, p1, (ninf, ninf), unroll=16)

        def tau_of(e1, e2):
            lo = jnp.minimum(_sortd(e1), _rev(_sortd(e2)))
            return lax.reduce_max(jnp.where(lane3, _sortd(lo), _NEGINF),
                                  axes=(0,))

        def pass2(rl, tau):
            base = lanes + rl * _S

            @plsc.parallel_loop(0, _NVEC, unroll=16, carry=base)
            def oidx(i, oidx):
                v = rowbuf[rl, pl.ds(i * _LANES, _LANES)]
                m = v >= tau
                plsc.store_scatter(cand, [oidx], v, mask=m)
                return oidx + jnp.where(m, i16, izero)

            return lax.shift_right_logical(oidx - base, 4)

        def merge1(s, rl, t0, t1, cnt):
            v = jnp.where(cnt > s, cand[pl.ds(rl * _S + s * _LANES, _LANES)],
                          _NEGINF)
            sv = _sortd(v)
            a = _sortd(jnp.maximum(t1, _rev(sv)))
            ra = _rev(a)
            return _sortd(jnp.maximum(t0, ra)), _sortd(jnp.minimum(t0, ra))

        def hinge(rl, t0, t1):
            d = _MARGIN - posv[rl]
            return (m0c * jnp.maximum(d + t0, 0.0)
                    + m1c * jnp.maximum(d + t1, 0.0))

        es = [pass1(rl) for rl in range(_CH)]
        taus = [tau_of(e1, e2) for (e1, e2) in es]
        cnts = [pass2(rl, taus[rl]) for rl in range(_CH)]
        cmax = cnts[0]
        for rl in range(1, _CH):
            cmax = jnp.maximum(cmax, cnts[rl])
        nmax = lax.reduce_max(cmax, axes=(0,))

        def mrg8(s, ts):
            return tuple(
                v for rl in range(_CH)
                for v in merge1(s, rl, ts[2 * rl], ts[2 * rl + 1], cnts[rl]))

        ts = lax.fori_loop(0, nmax, mrg8, (ninf,) * (2 * _CH))
        for rl in range(_CH):
            acc = acc + hinge(rl, ts[2 * rl], ts[2 * rl + 1])
        return acc

    nchunk = npairs * 2
    pltpu.async_copy(table.at[idx_v.at[pl.ds(0, _CH)]], rowbuf_a, sem_a)
    pltpu.async_copy(table.at[idx_v.at[pl.ds(_CH, _CH)]], rowbuf_b, sem_b)

    def pair_body(cc, acc):
        c0 = cc * 2
        c1 = c0 + 1
        pltpu.make_async_copy(table.at[idx_v.at[pl.ds(c0 * _CH, _CH)]],
                              rowbuf_a, sem_a).wait()
        acc = process_chunk(rowbuf_a, c0, acc)

        @pl.when(c0 + 2 < nchunk)
        def _():
            pltpu.async_copy(table.at[idx_v.at[pl.ds((c0 + 2) * _CH, _CH)]],
                             rowbuf_a, sem_a)

        pltpu.make_async_copy(table.at[idx_v.at[pl.ds(c1 * _CH, _CH)]],
                              rowbuf_b, sem_b).wait()
        acc = process_chunk(rowbuf_b, c1, acc)

        @pl.when(c1 + 2 < nchunk)
        def _():
            pltpu.async_copy(table.at[idx_v.at[pl.ds((c1 + 2) * _CH, _CH)]],
                             rowbuf_b, sem_b)

        return acc

    acc = lax.fori_loop(0, npairs, pair_body,
                        jnp.zeros((_LANES,), jnp.float32))
    accv[...] = acc * _SCALE
    pltpu.sync_copy(accv, out.at[wid])


_triplet_sc = functools.partial(
    pl.kernel,
    out_type=jax.ShapeDtypeStruct((_NW, _LANES), jnp.float32),
    mesh=plsc.VectorSubcoreMesh(
        core_axis_name="c", subcore_axis_name="s",
        num_cores=_NC, num_subcores=_NS),
    scratch_types=[
        pltpu.VMEM((_RPW0,), jnp.int32),
        pltpu.VMEM((_RPW0 + _LANES,), jnp.int32),
        pltpu.VMEM((_CH, _S), jnp.float32),
        pltpu.VMEM((_CH, _S), jnp.float32),
        pltpu.VMEM((_CH * _S,), jnp.float32),
        pltpu.VMEM((_LANES,), jnp.float32),
        pltpu.VMEM((2, _LANES), jnp.float32),
        pltpu.SemaphoreType.DMA,
        pltpu.SemaphoreType.DMA,
    ],
    compiler_params=pltpu.CompilerParams(needs_layout_passes=False),
)(_body)


def kernel(sim_matrix, b_ids, i_ids, j_ids):
    table = sim_matrix.reshape(_B * _L, _S)
    pad = jnp.zeros((_PPAD - _P,), jnp.int32)
    rowidx = jnp.concatenate([(b_ids * _L + i_ids).astype(jnp.int32), pad])
    jr = jnp.concatenate([j_ids.astype(jnp.int32), pad])
    perm = jax.random.permutation(jax.random.key(42), 2 * _N_NEG)[:_N_NEG]
    masks = (jnp.arange(2 * _LANES)[None, :] == perm[:, None]).astype(
        jnp.float32).sum(axis=0).reshape(2, _LANES)
    out = _triplet_sc(table, rowidx, jr, masks)
    return jnp.sum(out)


# speedup vs baseline: 1.1070x; 1.0279x over previous
import functools

import jax
import jax.numpy as jnp
from jax import lax
from jax.experimental import pallas as pl
from jax.experimental.pallas import tpu as pltpu
from jax.experimental.pallas import tpu_sc as plsc

_MARGIN = 1.0
_N_NEG = 10
_B, _L, _S = 4, 4096, 4096
_P = 8192
_LANES = 16
_NC, _NS = 2, 16
_NW = _NC * _NS
_NVEC = _S // _LANES
_CH = 8
_RPW0 = 256
_RPW1 = 256
_NEGINF = -3.0e38
_MASKVAL = -1000000000.0
_SCALE = 1.0 / (_P * _N_NEG)


def _sortd(x):
    k, _ = plsc.sort_key_val(x, x, descending=True)
    return k


def _rev(x):
    return lax.rev(x, (0,))


def _body(table, bids, iids, jids, mref, out, idx_v, jv, rowbuf_a, rowbuf_b,
          cand, accv, m_v, sem_a, sem_b):
    cid = lax.axis_index("c")
    sid = lax.axis_index("s")
    wid = sid * _NC + cid
    base = jnp.where(cid == 1, sid * _RPW0, _NS * _RPW0 + sid * _RPW1)
    npairs = jnp.where(cid == 1, _RPW0 // (2 * _CH), _RPW1 // (2 * _CH))
    pltpu.sync_copy(bids.at[pl.ds(base, _RPW0)], jv.at[pl.ds(0, _RPW0)])
    pltpu.sync_copy(iids.at[pl.ds(base, _RPW0)], idx_v)
    for k in range(_RPW0 // _LANES):
        sl = pl.ds(k * _LANES, _LANES)
        idx_v[sl] = jv[sl] * _L + idx_v[sl]
    pltpu.sync_copy(jids.at[pl.ds(base, _RPW0)], jv.at[pl.ds(0, _RPW0)])
    pltpu.sync_copy(mref, m_v)

    ninf = jnp.full((_LANES,), _NEGINF, jnp.float32)
    izero = jnp.zeros((_LANES,), jnp.int32)
    ione = jnp.full((_LANES,), 1, jnp.int32)
    i16 = jnp.full((_LANES,), _LANES, jnp.int32)
    lanes = lax.iota(jnp.int32, _LANES)
    lane3 = lanes == 3
    row8 = lanes < _CH
    m0c = m_v[0]
    m1c = m_v[1]

    def process_chunk(rowbuf, c, acc):
        jvec = jv[pl.ds(c * _CH, _LANES)]
        posv = plsc.load_gather(rowbuf, [lanes, jvec], mask=row8)
        plsc.store_scatter(rowbuf, [lanes, jvec],
                           jnp.full((_LANES,), _MASKVAL, jnp.float32),
                           mask=row8)

        def pass1(rl):
            def p1(i, c2):
                e1, e2 = c2
                v = rowbuf[rl, pl.ds(i * _LANES, _LANES)]
                e2 = jnp.maximum(e2, jnp.minimum(e1, v))
                e1 = jnp.maximum(e1, v)
                return e1, e2

            return lax.fori_loop(0, _NVEC, p1, (ninf, ninf), unroll=16)

        def tau_of(e1, e2):
            lo = jnp.minimum(_sortd(e1), _rev(_sortd(e2)))
            return lax.reduce_max(jnp.where(lane3, _sortd(lo), _NEGINF),
                                  axes=(0,))

        def pass2(rl, tau):
            base = lanes + rl * _S

            @plsc.parallel_loop(0, _NVEC, unroll=16, carry=base)
            def oidx(i, oidx):
                v = rowbuf[rl, pl.ds(i * _LANES, _LANES)]
                m = v >= tau
                plsc.store_scatter(cand, [oidx], v, mask=m)
                return oidx + jnp.where(m, i16, izero)

            return lax.shift_right_logical(oidx - base, 4)

        def merge1(s, rl, t0, t1, cnt):
            v = jnp.where(cnt > s, cand[pl.ds(rl * _S + s * _LANES, _LANES)],
                          _NEGINF)
            sv = _sortd(v)
            a = _sortd(jnp.maximum(t1, _rev(sv)))
            ra = _rev(a)
            return _sortd(jnp.maximum(t0, ra)), _sortd(jnp.minimum(t0, ra))

        def hinge(rl, t0, t1):
            d = _MARGIN - posv[rl]
            return (m0c * jnp.maximum(d + t0, 0.0)
                    + m1c * jnp.maximum(d + t1, 0.0))

        es = [pass1(rl) for rl in range(_CH)]
        taus = [tau_of(e1, e2) for (e1, e2) in es]
        cnts = [pass2(rl, taus[rl]) for rl in range(_CH)]
        cmax = cnts[0]
        for rl in range(1, _CH):
            cmax = jnp.maximum(cmax, cnts[rl])
        nmax = lax.reduce_max(cmax, axes=(0,))

        def mrg8(s, ts):
            return tuple(
                v for rl in range(_CH)
                for v in merge1(s, rl, ts[2 * rl], ts[2 * rl + 1], cnts[rl]))

        ts = lax.fori_loop(0, nmax, mrg8, (ninf,) * (2 * _CH))
        for rl in range(_CH):
            acc = acc + hinge(rl, ts[2 * rl], ts[2 * rl + 1])
        return acc

    nchunk = npairs * 2
    pltpu.async_copy(table.at[idx_v.at[pl.ds(0, _CH)]], rowbuf_a, sem_a)
    pltpu.async_copy(table.at[idx_v.at[pl.ds(_CH, _CH)]], rowbuf_b, sem_b)

    def pair_body(cc, acc):
        c0 = cc * 2
        c1 = c0 + 1
        pltpu.make_async_copy(table.at[idx_v.at[pl.ds(c0 * _CH, _CH)]],
                              rowbuf_a, sem_a).wait()
        acc = process_chunk(rowbuf_a, c0, acc)

        @pl.when(c0 + 2 < nchunk)
        def _():
            pltpu.async_copy(table.at[idx_v.at[pl.ds((c0 + 2) * _CH, _CH)]],
                             rowbuf_a, sem_a)

        pltpu.make_async_copy(table.at[idx_v.at[pl.ds(c1 * _CH, _CH)]],
                              rowbuf_b, sem_b).wait()
        acc = process_chunk(rowbuf_b, c1, acc)

        @pl.when(c1 + 2 < nchunk)
        def _():
            pltpu.async_copy(table.at[idx_v.at[pl.ds((c1 + 2) * _CH, _CH)]],
                             rowbuf_b, sem_b)

        return acc

    acc = lax.fori_loop(0, npairs, pair_body,
                        jnp.zeros((_LANES,), jnp.float32))
    accv[...] = acc * _SCALE
    pltpu.sync_copy(accv, out.at[wid])


_triplet_sc = functools.partial(
    pl.kernel,
    out_type=jax.ShapeDtypeStruct((_NW, _LANES), jnp.float32),
    mesh=plsc.VectorSubcoreMesh(
        core_axis_name="c", subcore_axis_name="s",
        num_cores=_NC, num_subcores=_NS),
    scratch_types=[
        pltpu.VMEM((_RPW0,), jnp.int32),
        pltpu.VMEM((_RPW0 + _LANES,), jnp.int32),
        pltpu.VMEM((_CH, _S), jnp.float32),
        pltpu.VMEM((_CH, _S), jnp.float32),
        pltpu.VMEM((_CH * _S,), jnp.float32),
        pltpu.VMEM((_LANES,), jnp.float32),
        pltpu.VMEM((2, _LANES), jnp.float32),
        pltpu.SemaphoreType.DMA,
        pltpu.SemaphoreType.DMA,
    ],
    compiler_params=pltpu.CompilerParams(needs_layout_passes=False),
)(_body)


def kernel(sim_matrix, b_ids, i_ids, j_ids):
    table = sim_matrix.reshape(_B * _L, _S)
    perm = jax.random.permutation(jax.random.key(42), 2 * _N_NEG)[:_N_NEG]
    masks = (jnp.arange(2 * _LANES)[None, :] == perm[:, None]).astype(
        jnp.float32).sum(axis=0).reshape(2, _LANES)
    out = _triplet_sc(table, b_ids, i_ids, j_ids, masks)
    return jnp.sum(out)
